# Initial kernel scaffold; baseline (speedup 1.0000x reference)
import jax
import jax.numpy as jnp
from jax.experimental import pallas as pl


def kernel(query, reference_points, value, spatial_shapes, level_start_index, Wv, bv, Ws_off, bs_off, Wa, ba, Wo, bo):
    raise NotImplementedError("write your pallas kernel here")


# speedup vs baseline: 1.0000x; 1.0000x over previous
import functools

import jax
import jax.numpy as jnp
import numpy as np
from jax.experimental import pallas as pl

B = 2
LQ = 13294
D_MODEL = 256
N_LEVELS = 4
N_HEADS = 8
N_POINTS = 4
D_HEAD = D_MODEL // N_HEADS
SPATIAL = np.array([[100, 100], [50, 50], [25, 25], [13, 13]], dtype=np.int64)
LEN_IN = int((SPATIAL[:, 0] * SPATIAL[:, 1]).sum())

MBLK = 512


def _mm_bias_kernel(x_ref, w_ref, b_ref, o_ref):
    o_ref[...] = (
        jnp.dot(x_ref[...], w_ref[...], preferred_element_type=jnp.float32)
        + b_ref[...]
    )


def _mm_bias(x, w, b):
    M, K = x.shape
    N = w.shape[1]
    return pl.pallas_call(
        _mm_bias_kernel,
        grid=(M // MBLK,),
        in_specs=[
            pl.BlockSpec((MBLK, K), lambda i: (i, 0)),
            pl.BlockSpec((K, N), lambda i: (0, 0)),
            pl.BlockSpec((1, N), lambda i: (0, 0)),
        ],
        out_specs=pl.BlockSpec((MBLK, N), lambda i: (i, 0)),
        out_shape=jax.ShapeDtypeStruct((M, N), jnp.float32),
    )(x, w, b.reshape(1, N))


def _bilinear(value_l, x, y, H_l, W_l, BH, Lq, P):
    def corner(xi, yi):
        valid = ((xi >= 0) & (xi < W_l) & (yi >= 0) & (yi < H_l)).astype(value_l.dtype)
        xi_c = jnp.clip(xi, 0, W_l - 1).astype(jnp.int32)
        yi_c = jnp.clip(yi, 0, H_l - 1).astype(jnp.int32)
        idx = (yi_c * W_l + xi_c).reshape(BH, 1, Lq * P)
        g = jnp.take_along_axis(
            value_l, jnp.broadcast_to(idx, (BH, value_l.shape[1], Lq * P)), axis=2
        )
        g = g.reshape(BH, -1, Lq, P)
        return g * valid.reshape(BH, 1, Lq, P)

    x0 = jnp.floor(x)
    y0 = jnp.floor(y)
    x1 = x0 + 1.0
    y1 = y0 + 1.0
    wx1 = x - x0
    wx0 = 1.0 - wx1
    wy1 = y - y0
    wy0 = 1.0 - wy1
    out = (
        corner(x0, y0) * (wx0 * wy0).reshape(BH, 1, Lq, P)
        + corner(x1, y0) * (wx1 * wy0).reshape(BH, 1, Lq, P)
        + corner(x0, y1) * (wx0 * wy1).reshape(BH, 1, Lq, P)
        + corner(x1, y1) * (wx1 * wy1).reshape(BH, 1, Lq, P)
    )
    return out


@jax.jit
def _run(query, reference_points, value, Wv, bv, Ws_off, bs_off, Wa, ba, Wo, bo):
    Bq, Lq, d = query.shape
    LQ_PAD = ((Lq + MBLK - 1) // MBLK) * MBLK
    LEN_PAD = ((LEN_IN + MBLK - 1) // MBLK) * MBLK

    q_pad = jnp.pad(query, ((0, 0), (0, LQ_PAD - Lq), (0, 0)))
    v_pad = jnp.pad(value, ((0, 0), (0, LEN_PAD - LEN_IN), (0, 0)))

    v = _mm_bias(v_pad.reshape(Bq * LEN_PAD, d), Wv, bv).reshape(Bq, LEN_PAD, d)
    v = v[:, :LEN_IN]
    off = _mm_bias(q_pad.reshape(Bq * LQ_PAD, d), Ws_off, bs_off)
    off = off.reshape(Bq, LQ_PAD, -1)[:, :Lq].reshape(
        Bq, Lq, N_HEADS, N_LEVELS, N_POINTS, 2
    )
    aw = _mm_bias(q_pad.reshape(Bq * LQ_PAD, d), Wa, ba)
    aw = aw.reshape(Bq, LQ_PAD, -1)[:, :Lq].reshape(Bq, Lq, N_HEADS, N_LEVELS * N_POINTS)
    aw = jax.nn.softmax(aw, axis=-1).reshape(Bq, Lq, N_HEADS, N_LEVELS, N_POINTS)

    lsi = np.concatenate([[0], np.cumsum(SPATIAL[:, 0] * SPATIAL[:, 1])[:-1]])
    wh = jnp.asarray(SPATIAL[:, [1, 0]], dtype=jnp.float32)
    loc = reference_points[:, :, None, :, None, :] + off / wh[None, None, None, :, None, :]
    BH = Bq * N_HEADS
    output = jnp.zeros((Bq, Lq, N_HEADS, D_HEAD), dtype=v.dtype)
    for lvl in range(N_LEVELS):
        H_l = int(SPATIAL[lvl, 0])
        W_l = int(SPATIAL[lvl, 1])
        value_l = jax.lax.dynamic_slice_in_dim(v, int(lsi[lvl]), H_l * W_l, axis=1)
        value_l = value_l.reshape(Bq, H_l, W_l, N_HEADS, D_HEAD)
        value_l = value_l.transpose(0, 3, 4, 1, 2).reshape(BH, D_HEAD, H_l * W_l)
        grid = 2.0 * loc[:, :, :, lvl] - 1.0
        grid = grid.transpose(0, 2, 1, 3, 4).reshape(BH, Lq, N_POINTS, 2)
        x = ((grid[..., 0] + 1.0) * W_l - 1.0) / 2.0
        y = ((grid[..., 1] + 1.0) * H_l - 1.0) / 2.0
        sampled = _bilinear(value_l, x, y, H_l, W_l, BH, Lq, N_POINTS)
        sampled = sampled.reshape(Bq, N_HEADS, D_HEAD, Lq, N_POINTS).transpose(0, 3, 1, 4, 2)
        output = output + (sampled * aw[:, :, :, lvl][..., None]).sum(-2)

    out = output.reshape(Bq, Lq, d)
    out_pad = jnp.pad(out, ((0, 0), (0, LQ_PAD - Lq), (0, 0)))
    res = _mm_bias(out_pad.reshape(Bq * LQ_PAD, d), Wo, bo)
    return res.reshape(Bq, LQ_PAD, d)[:, :Lq]


def kernel(query, reference_points, value, spatial_shapes, level_start_index,
           Wv, bv, Ws_off, bs_off, Wa, ba, Wo, bo):
    return _run(query, reference_points, value, Wv, bv, Ws_off, bs_off, Wa, ba, Wo, bo)


# speedup vs baseline: 1956.6567x; 1956.5706x over previous
import functools

import jax
import jax.numpy as jnp
import numpy as np
from jax import lax
from jax.experimental import pallas as pl
from jax.experimental.pallas import tpu as pltpu
from jax.experimental.pallas import tpu_sc as plsc

B = 2
LQ = 13294
D_MODEL = 256
N_LEVELS = 4
N_HEADS = 8
N_POINTS = 4
D_HEAD = 32
SPATIAL = np.array([[100, 100], [50, 50], [25, 25], [13, 13]], dtype=np.int64)
LEN_IN = int((SPATIAL[:, 0] * SPATIAL[:, 1]).sum())

MBLK = 512
QPAD = 13312
LENPAD = 13312
NQB = QPAD // MBLK
NTEC = 32
QPT = QPAD * B // NTEC

_lane = np.arange(128)
_h_of = _lane // 16
_l_of = (_lane // 4) % 4
_Wl = SPATIAL[_l_of, 1].astype(np.float32)
_Hl = SPATIAL[_l_of, 0].astype(np.float32)
_lsi = np.concatenate([[0], np.cumsum(SPATIAL[:, 0] * SPATIAL[:, 1])[:-1]])
_lsi_lane = _lsi[_l_of].astype(np.float32)
_h_lane = _h_of.astype(np.float32)

_SXW = np.zeros((8, 128), np.float32)
_SYH = np.zeros((8, 128), np.float32)
for _ln in range(128):
    _SXW[2 * _l_of[_ln], _ln] = _Wl[_ln]
    _SYH[2 * _l_of[_ln] + 1, _ln] = _Hl[_ln]

_BD = (_lane[:, None] // 16 == _lane[None, :] // 16).astype(np.float32)


def _mm_bias_kernel(x_ref, w_ref, b_ref, o_ref):
    o_ref[...] = (
        jnp.dot(x_ref[...], w_ref[...], preferred_element_type=jnp.float32, precision=lax.Precision.HIGHEST)
        + b_ref[...]
    )


def _mm_bias(x, w, b):
    M, K = x.shape
    N = w.shape[1]
    return pl.pallas_call(
        _mm_bias_kernel,
        grid=(M // MBLK,),
        in_specs=[
            pl.BlockSpec((MBLK, K), lambda i: (i, 0)),
            pl.BlockSpec((K, N), lambda i: (0, 0)),
            pl.BlockSpec((1, N), lambda i: (0, 0)),
        ],
        out_specs=pl.BlockSpec((MBLK, N), lambda i: (i, 0)),
        out_shape=jax.ShapeDtypeStruct((M, N), jnp.float32),
    )(x, w, b.reshape(1, N))


def _prep_kernel(q_ref, rp_ref, wsx_ref, wsy_ref, wa_ref, bd_ref, sxw_ref,
                 syh_ref, bsx_ref, bsy_ref, ba_ref, wl_ref, hl_ref, lsi_ref,
                 hlane_ref, idx_ref, coef_ref):
    q = q_ref[0]
    offx = jnp.dot(q, wsx_ref[...], preferred_element_type=jnp.float32, precision=lax.Precision.HIGHEST) + bsx_ref[...]
    offy = jnp.dot(q, wsy_ref[...], preferred_element_type=jnp.float32, precision=lax.Precision.HIGHEST) + bsy_ref[...]
    a = jnp.dot(q, wa_ref[...], preferred_element_type=jnp.float32, precision=lax.Precision.HIGHEST) + ba_ref[...]
    m = jnp.max(a, axis=1, keepdims=True)
    e = jnp.exp(a - m)
    s = jnp.dot(e, bd_ref[...], preferred_element_type=jnp.float32, precision=lax.Precision.HIGHEST)
    aw = e / s

    rp = rp_ref[0]
    x = jnp.dot(rp, sxw_ref[...], preferred_element_type=jnp.float32, precision=lax.Precision.HIGHEST) + offx - 0.5
    y = jnp.dot(rp, syh_ref[...], preferred_element_type=jnp.float32, precision=lax.Precision.HIGHEST) + offy - 0.5

    wl = wl_ref[...]
    hl = hl_ref[...]
    x0 = jnp.floor(x)
    y0 = jnp.floor(y)
    fx = x - x0
    fy = y - y0

    b_f = pl.program_id(0).astype(jnp.float32)
    base = b_f * (LENPAD * 8.0) + hlane_ref[...]

    for cy in range(2):
        yi = y0 + cy
        wy = fy if cy else 1.0 - fy
        vy = (yi >= 0.0) & (yi <= hl - 1.0)
        yc = jnp.clip(yi, 0.0, hl - 1.0)
        for cx in range(2):
            xi = x0 + cx
            wx = fx if cx else 1.0 - fx
            vx = (xi >= 0.0) & (xi <= wl - 1.0)
            xc = jnp.clip(xi, 0.0, wl - 1.0)
            pos = lsi_ref[...] + yc * wl + xc
            idxf = base + pos * 8.0
            coef = aw * wx * wy * jnp.where(vx & vy, 1.0, 0.0)
            c = cy * 2 + cx
            idx_ref[0, :, c, :] = idxf.astype(jnp.int32)
            coef_ref[0, :, c, :] = coef


def _prep(q_pad, rp_pad, wsx, wsy, wa, bsx, bsy, ba):
    small = [
        jnp.asarray(_BD), jnp.asarray(_SXW), jnp.asarray(_SYH),
        bsx.reshape(1, 128), bsy.reshape(1, 128), ba.reshape(1, 128),
        jnp.asarray(_Wl).reshape(1, 128), jnp.asarray(_Hl).reshape(1, 128),
        jnp.asarray(_lsi_lane).reshape(1, 128),
        jnp.asarray(_h_lane).reshape(1, 128),
    ]
    full = lambda shape: pl.BlockSpec(shape, lambda b, i: tuple(0 for _ in shape))
    return pl.pallas_call(
        _prep_kernel,
        grid=(B, NQB),
        in_specs=[
            pl.BlockSpec((1, MBLK, D_MODEL), lambda b, i: (b, i, 0)),
            pl.BlockSpec((1, MBLK, 8), lambda b, i: (b, i, 0)),
            full((D_MODEL, 128)),
            full((D_MODEL, 128)),
            full((D_MODEL, 128)),
            full((128, 128)),
            full((8, 128)),
            full((8, 128)),
            full((1, 128)),
            full((1, 128)),
            full((1, 128)),
            full((1, 128)),
            full((1, 128)),
            full((1, 128)),
            full((1, 128)),
        ],
        out_specs=[
            pl.BlockSpec((1, MBLK, 4, 128), lambda b, i: (b, i, 0, 0)),
            pl.BlockSpec((1, MBLK, 4, 128), lambda b, i: (b, i, 0, 0)),
        ],
        out_shape=[
            jax.ShapeDtypeStruct((B, QPAD, 4, 128), jnp.int32),
            jax.ShapeDtypeStruct((B, QPAD, 4, 128), jnp.float32),
        ],
    )(q_pad, rp_pad, wsx, wsy, wa, *small)


def _bcast_lane(v, j):
    return lax.gather(
        v,
        jnp.full((16, 1), j, jnp.int32),
        lax.GatherDimensionNumbers(
            offset_dims=(), collapsed_slice_dims=(0,), start_index_map=(0,)
        ),
        (1,),
        mode=lax.GatherScatterMode.PROMISE_IN_BOUNDS,
    )


def _sc_sample_body(vt_hbm, idx_hbm, coef_hbm, out_hbm, idx_v, coef_v, g_v,
                    out_v, sem):
    wid = lax.axis_index("s") * 2 + lax.axis_index("c")
    qbase = wid * QPT

    @pl.loop(0, QPT)
    def _q(qi):
        qrow = qbase + qi
        pltpu.sync_copy(idx_hbm.at[qrow], idx_v)
        pltpu.sync_copy(coef_hbm.at[qrow], coef_v)
        cps = [
            pltpu.async_copy(
                vt_hbm.at[idx_v.at[c]], g_v.at[pl.ds(c * 128, 128)], sem
            )
            for c in range(4)
        ]
        for cp in cps:
            cp.wait()

        @pl.loop(0, N_HEADS)
        def _h(h):
            h16 = h * 16
            acc0 = jnp.zeros((16,), jnp.float32)
            acc1 = jnp.zeros((16,), jnp.float32)
            for c in range(4):
                cv = coef_v[c, pl.ds(h16, 16)]
                for j in range(16):
                    w = _bcast_lane(cv, j)
                    row = h16 + (c * 128 + j)
                    acc0 = acc0 + w * g_v[row, pl.ds(0, 16)]
                    acc1 = acc1 + w * g_v[row, pl.ds(16, 16)]
            out_v[h, pl.ds(0, 16)] = acc0
            out_v[h, pl.ds(16, 16)] = acc1

        pltpu.sync_copy(out_v, out_hbm.at[pl.ds(qrow * N_HEADS, N_HEADS)])


@functools.cache
def _sc_sample():
    mesh = plsc.VectorSubcoreMesh(
        core_axis_name="c", subcore_axis_name="s", num_cores=2, num_subcores=16
    )
    return pl.kernel(
        _sc_sample_body,
        out_type=jax.ShapeDtypeStruct((B * QPAD * N_HEADS, D_HEAD), jnp.float32),
        mesh=mesh,
        scratch_types=[
            pltpu.VMEM((4, 128), jnp.int32),
            pltpu.VMEM((4, 128), jnp.float32),
            pltpu.VMEM((512, D_HEAD), jnp.float32),
            pltpu.VMEM((N_HEADS, D_HEAD), jnp.float32),
            pltpu.SemaphoreType.DMA,
        ],
        compiler_params=pltpu.CompilerParams(use_tc_tiling_on_sc=False),
    )


@jax.jit
def _run(query, reference_points, value, Wv, bv, Ws_off, bs_off, Wa, ba, Wo, bo):
    q_pad = jnp.pad(query, ((0, 0), (0, QPAD - LQ), (0, 0)))
    v_pad = jnp.pad(value, ((0, 0), (0, LENPAD - LEN_IN), (0, 0)))
    rp_pad = jnp.pad(reference_points, ((0, 0), (0, QPAD - LQ), (0, 0), (0, 0)))
    rp_pad = rp_pad.reshape(B, QPAD, 8)

    ws = Ws_off.reshape(D_MODEL, N_HEADS, N_LEVELS, N_POINTS, 2)
    wsx = ws[..., 0].reshape(D_MODEL, 128)
    wsy = ws[..., 1].reshape(D_MODEL, 128)
    bs = bs_off.reshape(N_HEADS, N_LEVELS, N_POINTS, 2)
    bsx = bs[..., 0].reshape(128)
    bsy = bs[..., 1].reshape(128)

    vt = _mm_bias(v_pad.reshape(B * LENPAD, D_MODEL), Wv, bv)
    vt = vt.reshape(B * LENPAD * N_HEADS, D_HEAD)

    idx4, coef4 = _prep(q_pad, rp_pad, wsx, wsy, Wa, bsx, bsy, ba)
    idx3 = idx4.reshape(B * QPAD, 4, 128)
    coef3 = coef4.reshape(B * QPAD, 4, 128)

    attn = _sc_sample()(vt, idx3, coef3)
    attn = attn.reshape(B * QPAD, D_MODEL)

    out = _mm_bias(attn, Wo, bo)
    return out.reshape(B, QPAD, D_MODEL)[:, :LQ]


def kernel(query, reference_points, value, spatial_shapes, level_start_index,
           Wv, bv, Ws_off, bs_off, Wa, ba, Wo, bo):
    return _run(query, reference_points, value, Wv, bv, Ws_off, bs_off,
                Wa, ba, Wo, bo)


# speedup vs baseline: 3034.2321x; 1.5507x over previous
import functools

import jax
import jax.numpy as jnp
import numpy as np
from jax import lax
from jax.experimental import pallas as pl
from jax.experimental.pallas import tpu as pltpu
from jax.experimental.pallas import tpu_sc as plsc

B = 2
LQ = 13294
D_MODEL = 256
N_LEVELS = 4
N_HEADS = 8
N_POINTS = 4
D_HEAD = 32
SPATIAL = np.array([[100, 100], [50, 50], [25, 25], [13, 13]], dtype=np.int64)
LEN_IN = int((SPATIAL[:, 0] * SPATIAL[:, 1]).sum())

MBLK = 512
QPAD = 13312
LENPAD = 13312
NQB = QPAD // MBLK
NTEC = 32
QPT = QPAD * B // NTEC

_lane = np.arange(128)
_h_of = _lane // 16
_l_of = (_lane // 4) % 4
_Wl = SPATIAL[_l_of, 1].astype(np.float32)
_Hl = SPATIAL[_l_of, 0].astype(np.float32)
_lsi = np.concatenate([[0], np.cumsum(SPATIAL[:, 0] * SPATIAL[:, 1])[:-1]])
_lsi_lane = _lsi[_l_of].astype(np.float32)
_h_lane = _h_of.astype(np.float32)

_SXW = np.zeros((8, 128), np.float32)
_SYH = np.zeros((8, 128), np.float32)
for _ln in range(128):
    _SXW[2 * _l_of[_ln], _ln] = _Wl[_ln]
    _SYH[2 * _l_of[_ln] + 1, _ln] = _Hl[_ln]

_BD = (_lane[:, None] // 16 == _lane[None, :] // 16).astype(np.float32)


def _mm_bias_kernel(x_ref, w_ref, b_ref, o_ref):
    o_ref[...] = (
        jnp.dot(x_ref[...], w_ref[...], preferred_element_type=jnp.float32, precision=lax.Precision.HIGHEST)
        + b_ref[...]
    )


def _mm_bias(x, w, b):
    M, K = x.shape
    N = w.shape[1]
    return pl.pallas_call(
        _mm_bias_kernel,
        grid=(M // MBLK,),
        in_specs=[
            pl.BlockSpec((MBLK, K), lambda i: (i, 0)),
            pl.BlockSpec((K, N), lambda i: (0, 0)),
            pl.BlockSpec((1, N), lambda i: (0, 0)),
        ],
        out_specs=pl.BlockSpec((MBLK, N), lambda i: (i, 0)),
        out_shape=jax.ShapeDtypeStruct((M, N), jnp.float32),
    )(x, w, b.reshape(1, N))


def _prep_kernel(q_ref, rp_ref, wsx_ref, wsy_ref, wa_ref, bd_ref, sxw_ref,
                 syh_ref, bsx_ref, bsy_ref, ba_ref, wl_ref, hl_ref, lsi_ref,
                 hlane_ref, idx_ref, coef_ref):
    q = q_ref[0]
    offx = jnp.dot(q, wsx_ref[...], preferred_element_type=jnp.float32, precision=lax.Precision.HIGHEST) + bsx_ref[...]
    offy = jnp.dot(q, wsy_ref[...], preferred_element_type=jnp.float32, precision=lax.Precision.HIGHEST) + bsy_ref[...]
    a = jnp.dot(q, wa_ref[...], preferred_element_type=jnp.float32, precision=lax.Precision.HIGHEST) + ba_ref[...]
    m = jnp.max(a, axis=1, keepdims=True)
    e = jnp.exp(a - m)
    s = jnp.dot(e, bd_ref[...], preferred_element_type=jnp.float32, precision=lax.Precision.HIGHEST)
    aw = e / s

    rp = rp_ref[0]
    x = jnp.dot(rp, sxw_ref[...], preferred_element_type=jnp.float32, precision=lax.Precision.HIGHEST) + offx - 0.5
    y = jnp.dot(rp, syh_ref[...], preferred_element_type=jnp.float32, precision=lax.Precision.HIGHEST) + offy - 0.5

    wl = wl_ref[...]
    hl = hl_ref[...]
    x0 = jnp.floor(x)
    y0 = jnp.floor(y)
    fx = x - x0
    fy = y - y0

    b_f = pl.program_id(0).astype(jnp.float32)
    base = b_f * (LENPAD * 8.0) + hlane_ref[...]

    for cy in range(2):
        yi = y0 + cy
        wy = fy if cy else 1.0 - fy
        vy = (yi >= 0.0) & (yi <= hl - 1.0)
        yc = jnp.clip(yi, 0.0, hl - 1.0)
        for cx in range(2):
            xi = x0 + cx
            wx = fx if cx else 1.0 - fx
            vx = (xi >= 0.0) & (xi <= wl - 1.0)
            xc = jnp.clip(xi, 0.0, wl - 1.0)
            pos = lsi_ref[...] + yc * wl + xc
            idxf = base + pos * 8.0
            coef = aw * wx * wy * jnp.where(vx & vy, 1.0, 0.0)
            c = cy * 2 + cx
            idx_ref[0, :, c, :] = idxf.astype(jnp.int32)
            coef_ref[0, :, c, :] = coef


def _prep(q_pad, rp_pad, wsx, wsy, wa, bsx, bsy, ba):
    small = [
        jnp.asarray(_BD), jnp.asarray(_SXW), jnp.asarray(_SYH),
        bsx.reshape(1, 128), bsy.reshape(1, 128), ba.reshape(1, 128),
        jnp.asarray(_Wl).reshape(1, 128), jnp.asarray(_Hl).reshape(1, 128),
        jnp.asarray(_lsi_lane).reshape(1, 128),
        jnp.asarray(_h_lane).reshape(1, 128),
    ]
    full = lambda shape: pl.BlockSpec(shape, lambda b, i: tuple(0 for _ in shape))
    return pl.pallas_call(
        _prep_kernel,
        grid=(B, NQB),
        in_specs=[
            pl.BlockSpec((1, MBLK, D_MODEL), lambda b, i: (b, i, 0)),
            pl.BlockSpec((1, MBLK, 8), lambda b, i: (b, i, 0)),
            full((D_MODEL, 128)),
            full((D_MODEL, 128)),
            full((D_MODEL, 128)),
            full((128, 128)),
            full((8, 128)),
            full((8, 128)),
            full((1, 128)),
            full((1, 128)),
            full((1, 128)),
            full((1, 128)),
            full((1, 128)),
            full((1, 128)),
            full((1, 128)),
        ],
        out_specs=[
            pl.BlockSpec((1, MBLK, 4, 128), lambda b, i: (b, i, 0, 0)),
            pl.BlockSpec((1, MBLK, 4, 128), lambda b, i: (b, i, 0, 0)),
        ],
        out_shape=[
            jax.ShapeDtypeStruct((B, QPAD, 4, 128), jnp.int32),
            jax.ShapeDtypeStruct((B, QPAD, 4, 128), jnp.float32),
        ],
    )(q_pad, rp_pad, wsx, wsy, wa, *small)


def _bcast_lane(v, j):
    return lax.gather(
        v,
        jnp.full((16, 1), j, jnp.int32),
        lax.GatherDimensionNumbers(
            offset_dims=(), collapsed_slice_dims=(0,), start_index_map=(0,)
        ),
        (1,),
        mode=lax.GatherScatterMode.PROMISE_IN_BOUNDS,
    )


CQ = 2
NCH = QPT // CQ
GROWS = CQ * 512


def _sc_sample_body(vt_hbm, idx_hbm, coef_hbm, out_hbm, idx0, idx1, coef0,
                    coef1, g0, g1, out_v, semi0, semi1, semg0, semg1):
    IV = (idx0, idx1)
    CV = (coef0, coef1)
    GV = (g0, g1)
    SI = (semi0, semi1)
    SG = (semg0, semg1)

    wid = lax.axis_index("s") * 2 + lax.axis_index("c")
    qbase = wid * QPT

    def fire_gather(p):
        for qq in range(CQ):
            for c in range(4):
                pltpu.async_copy(
                    vt_hbm.at[IV[p].at[qq, c]],
                    GV[p].at[pl.ds((qq * 4 + c) * 128, 128)],
                    SG[p],
                )

    def compute(p, gg):
        for qq in range(CQ):
            @pl.loop(0, N_HEADS)
            def _h(h):
                h16 = h * 16
                acc0 = jnp.zeros((16,), jnp.float32)
                acc1 = jnp.zeros((16,), jnp.float32)
                for c in range(4):
                    cv = CV[p][qq, c, pl.ds(h16, 16)]
                    for j in range(16):
                        w = _bcast_lane(cv, j)
                        row = h16 + (qq * 512 + c * 128 + j)
                        acc0 = acc0 + w * GV[p][row, pl.ds(0, 16)]
                        acc1 = acc1 + w * GV[p][row, pl.ds(16, 16)]
                out_v[qq * N_HEADS + h, pl.ds(0, 16)] = acc0
                out_v[qq * N_HEADS + h, pl.ds(16, 16)] = acc1

    pltpu.sync_copy(idx_hbm.at[pl.ds(qbase, CQ)], IV[0])
    pltpu.sync_copy(coef_hbm.at[pl.ds(qbase, CQ)], CV[0])
    fire_gather(0)

    @pl.loop(0, NCH, step=2)
    def _g(g):
        for p in range(2):
            gg = g + p
            nxt = gg + 1

            @pl.when(nxt < NCH)
            def _pref():
                qrow = qbase + nxt * CQ
                pltpu.async_copy(idx_hbm.at[pl.ds(qrow, CQ)], IV[1 - p], SI[1 - p])
                pltpu.async_copy(coef_hbm.at[pl.ds(qrow, CQ)], CV[1 - p], SI[1 - p])

            pltpu.make_async_copy(vt_hbm.at[pl.ds(0, GROWS)], GV[p], SG[p]).wait()

            compute(p, gg)

            @pl.when(nxt < NCH)
            def _fire():
                pltpu.make_async_copy(
                    idx_hbm.at[pl.ds(0, CQ)], IV[1 - p], SI[1 - p]
                ).wait()
                pltpu.make_async_copy(
                    coef_hbm.at[pl.ds(0, CQ)], CV[1 - p], SI[1 - p]
                ).wait()
                fire_gather(1 - p)

            pltpu.sync_copy(
                out_v, out_hbm.at[pl.ds((qbase + gg * CQ) * N_HEADS, CQ * N_HEADS)]
            )


@functools.cache
def _sc_sample():
    mesh = plsc.VectorSubcoreMesh(
        core_axis_name="c", subcore_axis_name="s", num_cores=2, num_subcores=16
    )
    return pl.kernel(
        _sc_sample_body,
        out_type=jax.ShapeDtypeStruct((B * QPAD * N_HEADS, D_HEAD), jnp.float32),
        mesh=mesh,
        scratch_types=[
            pltpu.VMEM((CQ, 4, 128), jnp.int32),
            pltpu.VMEM((CQ, 4, 128), jnp.int32),
            pltpu.VMEM((CQ, 4, 128), jnp.float32),
            pltpu.VMEM((CQ, 4, 128), jnp.float32),
            pltpu.VMEM((GROWS, D_HEAD), jnp.float32),
            pltpu.VMEM((GROWS, D_HEAD), jnp.float32),
            pltpu.VMEM((CQ * N_HEADS, D_HEAD), jnp.float32),
            pltpu.SemaphoreType.DMA,
            pltpu.SemaphoreType.DMA,
            pltpu.SemaphoreType.DMA,
            pltpu.SemaphoreType.DMA,
        ],
        compiler_params=pltpu.CompilerParams(use_tc_tiling_on_sc=False),
    )


@jax.jit
def _run(query, reference_points, value, Wv, bv, Ws_off, bs_off, Wa, ba, Wo, bo):
    q_pad = jnp.pad(query, ((0, 0), (0, QPAD - LQ), (0, 0)))
    v_pad = jnp.pad(value, ((0, 0), (0, LENPAD - LEN_IN), (0, 0)))
    rp_pad = jnp.pad(reference_points, ((0, 0), (0, QPAD - LQ), (0, 0), (0, 0)))
    rp_pad = rp_pad.reshape(B, QPAD, 8)

    ws = Ws_off.reshape(D_MODEL, N_HEADS, N_LEVELS, N_POINTS, 2)
    wsx = ws[..., 0].reshape(D_MODEL, 128)
    wsy = ws[..., 1].reshape(D_MODEL, 128)
    bs = bs_off.reshape(N_HEADS, N_LEVELS, N_POINTS, 2)
    bsx = bs[..., 0].reshape(128)
    bsy = bs[..., 1].reshape(128)

    vt = _mm_bias(v_pad.reshape(B * LENPAD, D_MODEL), Wv, bv)
    vt = vt.reshape(B * LENPAD * N_HEADS, D_HEAD)

    idx4, coef4 = _prep(q_pad, rp_pad, wsx, wsy, Wa, bsx, bsy, ba)
    idx3 = idx4.reshape(B * QPAD, 4, 128)
    coef3 = coef4.reshape(B * QPAD, 4, 128)

    attn = _sc_sample()(vt, idx3, coef3)
    attn = attn.reshape(B * QPAD, D_MODEL)

    out = _mm_bias(attn, Wo, bo)
    return out.reshape(B, QPAD, D_MODEL)[:, :LQ]


def kernel(query, reference_points, value, spatial_shapes, level_start_index,
           Wv, bv, Ws_off, bs_off, Wa, ba, Wo, bo):
    return _run(query, reference_points, value, Wv, bv, Ws_off, bs_off,
                Wa, ba, Wo, bo)


# speedup vs baseline: 3094.6308x; 1.0199x over previous
import functools

import jax
import jax.numpy as jnp
import numpy as np
from jax import lax
from jax.experimental import pallas as pl
from jax.experimental.pallas import tpu as pltpu
from jax.experimental.pallas import tpu_sc as plsc

B = 2
LQ = 13294
D_MODEL = 256
N_LEVELS = 4
N_HEADS = 8
N_POINTS = 4
D_HEAD = 32
SPATIAL = np.array([[100, 100], [50, 50], [25, 25], [13, 13]], dtype=np.int64)
LEN_IN = int((SPATIAL[:, 0] * SPATIAL[:, 1]).sum())

MBLK = 512
QPAD = 13312
LENPAD = 13312
NQB = QPAD // MBLK
NTEC = 32
QPT = QPAD * B // NTEC

_lane = np.arange(128)
_h_of = _lane // 16
_l_of = (_lane // 4) % 4
_Wl = SPATIAL[_l_of, 1].astype(np.float32)
_Hl = SPATIAL[_l_of, 0].astype(np.float32)
_lsi = np.concatenate([[0], np.cumsum(SPATIAL[:, 0] * SPATIAL[:, 1])[:-1]])
_lsi_lane = _lsi[_l_of].astype(np.float32)
_h_lane = _h_of.astype(np.float32)

_SXW = np.zeros((8, 128), np.float32)
_SYH = np.zeros((8, 128), np.float32)
for _ln in range(128):
    _SXW[2 * _l_of[_ln], _ln] = _Wl[_ln]
    _SYH[2 * _l_of[_ln] + 1, _ln] = _Hl[_ln]

_BD = (_lane[:, None] // 16 == _lane[None, :] // 16).astype(np.float32)


def _mm_bias_kernel(x_ref, w_ref, b_ref, o_ref):
    o_ref[...] = (
        jnp.dot(x_ref[...], w_ref[...], preferred_element_type=jnp.float32, precision=lax.Precision.HIGHEST)
        + b_ref[...]
    )


def _mm_bias(x, w, b):
    M, K = x.shape
    N = w.shape[1]
    return pl.pallas_call(
        _mm_bias_kernel,
        grid=(M // MBLK,),
        in_specs=[
            pl.BlockSpec((MBLK, K), lambda i: (i, 0)),
            pl.BlockSpec((K, N), lambda i: (0, 0)),
            pl.BlockSpec((1, N), lambda i: (0, 0)),
        ],
        out_specs=pl.BlockSpec((MBLK, N), lambda i: (i, 0)),
        out_shape=jax.ShapeDtypeStruct((M, N), jnp.float32),
    )(x, w, b.reshape(1, N))


def _prep_kernel(q_ref, rp_ref, wsx_ref, wsy_ref, wa_ref, bd_ref, sxw_ref,
                 syh_ref, bsx_ref, bsy_ref, ba_ref, wl_ref, hl_ref, lsi_ref,
                 hlane_ref, idx_ref, coef_ref):
    q = q_ref[0]
    offx = jnp.dot(q, wsx_ref[...], preferred_element_type=jnp.float32, precision=lax.Precision.HIGHEST) + bsx_ref[...]
    offy = jnp.dot(q, wsy_ref[...], preferred_element_type=jnp.float32, precision=lax.Precision.HIGHEST) + bsy_ref[...]
    a = jnp.dot(q, wa_ref[...], preferred_element_type=jnp.float32, precision=lax.Precision.HIGHEST) + ba_ref[...]
    m = jnp.max(a, axis=1, keepdims=True)
    e = jnp.exp(a - m)
    s = jnp.dot(e, bd_ref[...], preferred_element_type=jnp.float32, precision=lax.Precision.HIGHEST)
    aw = e / s

    rp = rp_ref[0]
    x = jnp.dot(rp, sxw_ref[...], preferred_element_type=jnp.float32, precision=lax.Precision.HIGHEST) + offx - 0.5
    y = jnp.dot(rp, syh_ref[...], preferred_element_type=jnp.float32, precision=lax.Precision.HIGHEST) + offy - 0.5

    wl = wl_ref[...]
    hl = hl_ref[...]
    x0 = jnp.floor(x)
    y0 = jnp.floor(y)
    fx = x - x0
    fy = y - y0

    b_f = pl.program_id(0).astype(jnp.float32)
    base = b_f * (LENPAD * 8.0) + hlane_ref[...]

    for cy in range(2):
        yi = y0 + cy
        wy = fy if cy else 1.0 - fy
        vy = (yi >= 0.0) & (yi <= hl - 1.0)
        yc = jnp.clip(yi, 0.0, hl - 1.0)
        for cx in range(2):
            xi = x0 + cx
            wx = fx if cx else 1.0 - fx
            vx = (xi >= 0.0) & (xi <= wl - 1.0)
            xc = jnp.clip(xi, 0.0, wl - 1.0)
            pos = lsi_ref[...] + yc * wl + xc
            idxf = base + pos * 8.0
            coef = aw * wx * wy * jnp.where(vx & vy, 1.0, 0.0)
            c = cy * 2 + cx
            idx_ref[0, c, :, :] = idxf.astype(jnp.int32)
            coef_ref[0, c, :, :] = coef


def _prep(q_pad, rp_pad, wsx, wsy, wa, bsx, bsy, ba):
    small = [
        jnp.asarray(_BD), jnp.asarray(_SXW), jnp.asarray(_SYH),
        bsx.reshape(1, 128), bsy.reshape(1, 128), ba.reshape(1, 128),
        jnp.asarray(_Wl).reshape(1, 128), jnp.asarray(_Hl).reshape(1, 128),
        jnp.asarray(_lsi_lane).reshape(1, 128),
        jnp.asarray(_h_lane).reshape(1, 128),
    ]
    full = lambda shape: pl.BlockSpec(shape, lambda b, i: tuple(0 for _ in shape))
    return pl.pallas_call(
        _prep_kernel,
        grid=(B, NQB),
        in_specs=[
            pl.BlockSpec((1, MBLK, D_MODEL), lambda b, i: (b, i, 0)),
            pl.BlockSpec((1, MBLK, 8), lambda b, i: (b, i, 0)),
            full((D_MODEL, 128)),
            full((D_MODEL, 128)),
            full((D_MODEL, 128)),
            full((128, 128)),
            full((8, 128)),
            full((8, 128)),
            full((1, 128)),
            full((1, 128)),
            full((1, 128)),
            full((1, 128)),
            full((1, 128)),
            full((1, 128)),
            full((1, 128)),
        ],
        out_specs=[
            pl.BlockSpec((1, 4, MBLK, 128), lambda b, i: (b, 0, i, 0)),
            pl.BlockSpec((1, 4, MBLK, 128), lambda b, i: (b, 0, i, 0)),
        ],
        out_shape=[
            jax.ShapeDtypeStruct((B, 4, QPAD, 128), jnp.int32),
            jax.ShapeDtypeStruct((B, 4, QPAD, 128), jnp.float32),
        ],
    )(q_pad, rp_pad, wsx, wsy, wa, *small)


def _bcast_lane(v, j):
    return lax.gather(
        v,
        jnp.full((16, 1), j, jnp.int32),
        lax.GatherDimensionNumbers(
            offset_dims=(), collapsed_slice_dims=(0,), start_index_map=(0,)
        ),
        (1,),
        mode=lax.GatherScatterMode.PROMISE_IN_BOUNDS,
    )


CQ = 2
NCH = QPT // CQ
GROWS = CQ * 512


def _sc_sample_body(vt_hbm, idx_hbm, coef_hbm, out_hbm, idx0, idx1, coef0,
                    coef1, g0, g1, out_v, semi0, semi1, semg0, semg1):
    IV = (idx0, idx1)
    CV = (coef0, coef1)
    GV = (g0, g1)
    SI = (semi0, semi1)
    SG = (semg0, semg1)

    wid = lax.axis_index("s") * 2 + lax.axis_index("c")
    qbase = wid * QPT
    b = wid // 16
    qloc0 = qbase - b * QPAD

    def fire_in(ch, p, sync):
        ql = qloc0 + ch * CQ
        for c in range(4):
            src_i = idx_hbm.at[b * 4 + c, pl.ds(ql, CQ)]
            src_c = coef_hbm.at[b * 4 + c, pl.ds(ql, CQ)]
            if sync:
                pltpu.sync_copy(src_i, IV[p].at[c])
                pltpu.sync_copy(src_c, CV[p].at[c])
            else:
                pltpu.async_copy(src_i, IV[p].at[c], SI[p])
                pltpu.async_copy(src_c, CV[p].at[c], SI[p])

    def drain_in(p):
        for c in range(4):
            pltpu.make_async_copy(
                idx_hbm.at[0, pl.ds(0, CQ)], IV[p].at[c], SI[p]
            ).wait()
            pltpu.make_async_copy(
                coef_hbm.at[0, pl.ds(0, CQ)], CV[p].at[c], SI[p]
            ).wait()

    def fire_gather(p):
        for qq in range(CQ):
            for c in range(4):
                pltpu.async_copy(
                    vt_hbm.at[IV[p].at[c, qq]],
                    GV[p].at[pl.ds((qq * 4 + c) * 128, 128)],
                    SG[p],
                )

    def compute(p):
        for qq in range(CQ):
            @pl.loop(0, N_HEADS)
            def _h(h):
                h16 = h * 16
                h32 = h * 32
                acc0 = jnp.zeros((16,), jnp.float32)
                acc1 = jnp.zeros((16,), jnp.float32)
                for c in range(4):
                    cv = CV[p][c, qq, pl.ds(h16, 16)]
                    for j in range(16):
                        w = _bcast_lane(cv, j)
                        row = h16 + (qq * 512 + c * 128 + j)
                        acc0 = acc0 + w * GV[p][row, pl.ds(0, 16)]
                        acc1 = acc1 + w * GV[p][row, pl.ds(16, 16)]
                out_v[qq, pl.ds(h32, 16)] = acc0
                out_v[qq, pl.ds(h32 + 16, 16)] = acc1

    fire_in(0, 0, True)
    fire_gather(0)

    @pl.loop(0, NCH, step=2)
    def _g(g):
        for p in range(2):
            gg = g + p
            nxt = gg + 1

            @pl.when(nxt < NCH)
            def _pref():
                fire_in(nxt, 1 - p, False)

            pltpu.make_async_copy(vt_hbm.at[pl.ds(0, GROWS)], GV[p], SG[p]).wait()

            compute(p)

            @pl.when(nxt < NCH)
            def _fire():
                drain_in(1 - p)
                fire_gather(1 - p)

            pltpu.sync_copy(out_v, out_hbm.at[pl.ds(qbase + gg * CQ, CQ)])


@functools.cache
def _sc_sample():
    mesh = plsc.VectorSubcoreMesh(
        core_axis_name="c", subcore_axis_name="s", num_cores=2, num_subcores=16
    )
    return pl.kernel(
        _sc_sample_body,
        out_type=jax.ShapeDtypeStruct((B * QPAD, D_MODEL), jnp.float32),
        mesh=mesh,
        scratch_types=[
            pltpu.VMEM((4, CQ, 128), jnp.int32),
            pltpu.VMEM((4, CQ, 128), jnp.int32),
            pltpu.VMEM((4, CQ, 128), jnp.float32),
            pltpu.VMEM((4, CQ, 128), jnp.float32),
            pltpu.VMEM((GROWS, D_HEAD), jnp.float32),
            pltpu.VMEM((GROWS, D_HEAD), jnp.float32),
            pltpu.VMEM((CQ, D_MODEL), jnp.float32),
            pltpu.SemaphoreType.DMA,
            pltpu.SemaphoreType.DMA,
            pltpu.SemaphoreType.DMA,
            pltpu.SemaphoreType.DMA,
        ],
        compiler_params=pltpu.CompilerParams(use_tc_tiling_on_sc=False),
    )


@jax.jit
def _run(query, reference_points, value, Wv, bv, Ws_off, bs_off, Wa, ba, Wo, bo):
    q_pad = jnp.pad(query, ((0, 0), (0, QPAD - LQ), (0, 0)))
    v_pad = jnp.pad(value, ((0, 0), (0, LENPAD - LEN_IN), (0, 0)))
    rp_pad = jnp.pad(reference_points, ((0, 0), (0, QPAD - LQ), (0, 0), (0, 0)))
    rp_pad = rp_pad.reshape(B, QPAD, 8)

    ws = Ws_off.reshape(D_MODEL, N_HEADS, N_LEVELS, N_POINTS, 2)
    wsx = ws[..., 0].reshape(D_MODEL, 128)
    wsy = ws[..., 1].reshape(D_MODEL, 128)
    bs = bs_off.reshape(N_HEADS, N_LEVELS, N_POINTS, 2)
    bsx = bs[..., 0].reshape(128)
    bsy = bs[..., 1].reshape(128)

    vt = _mm_bias(v_pad.reshape(B * LENPAD, D_MODEL), Wv, bv)
    vt = vt.reshape(B * LENPAD * N_HEADS, D_HEAD)

    idx4, coef4 = _prep(q_pad, rp_pad, wsx, wsy, Wa, bsx, bsy, ba)
    idx3 = idx4.reshape(B * 4, QPAD, 128)
    coef3 = coef4.reshape(B * 4, QPAD, 128)

    attn = _sc_sample()(vt, idx3, coef3)

    out = _mm_bias(attn, Wo, bo)
    return out.reshape(B, QPAD, D_MODEL)[:, :LQ]


def kernel(query, reference_points, value, spatial_shapes, level_start_index,
           Wv, bv, Ws_off, bs_off, Wa, ba, Wo, bo):
    return _run(query, reference_points, value, Wv, bv, Ws_off, bs_off,
                Wa, ba, Wo, bo)
